# Initial kernel scaffold; baseline (speedup 1.0000x reference)
import jax
import jax.numpy as jnp
from jax.experimental import pallas as pl


def kernel(ps, ns, zs):
    raise NotImplementedError("write your pallas kernel here")


# speedup vs baseline: 1.1487x; 1.1487x over previous
import functools

import jax
import jax.numpy as jnp
from jax import lax
from jax.experimental import pallas as pl
from jax.experimental.pallas import tpu as pltpu
from jax.experimental.pallas import tpu_sc as plsc

_NC, _NS, _L = 2, 16, 16
_NW = _NC * _NS
_B = 400


def _make_sc_decode(n_edges, n_rows, d):
    n_per_w = n_edges // _NW
    n_chunks = n_per_w // _B
    assert n_per_w % _B == 0 and _B % _L == 0
    n_groups = _B // _L
    mesh = plsc.VectorSubcoreMesh(core_axis_name="c", subcore_axis_name="s")

    @functools.partial(
        pl.kernel,
        out_type=jax.ShapeDtypeStruct((n_edges,), jnp.float32),
        mesh=mesh,
        scratch_types=[
            pltpu.VMEM((_B,), jnp.int32),
            pltpu.VMEM((_B,), jnp.int32),
            pltpu.VMEM((_B, d), jnp.float32),
            pltpu.VMEM((_B, d), jnp.float32),
            pltpu.VMEM((_B,), jnp.float32),
            pltpu.SemaphoreType.DMA,
            pltpu.SemaphoreType.DMA,
        ],
        compiler_params=pltpu.CompilerParams(needs_layout_passes=False),
    )
    def decode(src_hbm, dst_hbm, table_hbm, out_hbm,
               idx_s, idx_d, rows_s, rows_d, out_v, sem_s, sem_d):
        wid = lax.axis_index("s") * _NC + lax.axis_index("c")
        wbase = wid * n_per_w

        def chunk_body(c, carry):
            base = wbase + c * _B
            pltpu.sync_copy(src_hbm.at[pl.ds(base, _B)], idx_s)
            pltpu.sync_copy(dst_hbm.at[pl.ds(base, _B)], idx_d)
            cp_s = pltpu.async_copy(table_hbm.at[idx_s], rows_s, sem_s)
            cp_d = pltpu.async_copy(table_hbm.at[idx_d], rows_d, sem_d)
            cp_s.wait()
            cp_d.wait()

            def group_body(j, carry2):
                edge_ids = j * _L + lax.iota(jnp.int32, _L)

                def d_body(k, acc):
                    col = jnp.full((_L,), k, jnp.int32)
                    a = plsc.load_gather(rows_s, [edge_ids, col])
                    b = plsc.load_gather(rows_d, [edge_ids, col])
                    return acc + a * b

                acc = lax.fori_loop(0, d, d_body,
                                    jnp.zeros((_L,), jnp.float32), unroll=4)
                out_v[pl.ds(j * _L, _L)] = 1.0 / (1.0 + jnp.exp(-acc))
                return carry2

            lax.fori_loop(0, n_groups, group_body, 0)
            pltpu.sync_copy(out_v, out_hbm.at[pl.ds(base, _B)])
            return carry

        lax.fori_loop(0, n_chunks, chunk_body, 0)

    return decode


def kernel(ps, ns, zs):
    t, n, d = zs.shape
    e = ps.shape[2]
    s = t - 1
    off = (jnp.arange(s, dtype=jnp.int32) * n)[:, None]
    p_src = (ps[1:, 0].astype(jnp.int32) + off).reshape(-1)
    p_dst = (ps[1:, 1].astype(jnp.int32) + off).reshape(-1)
    n_src = (ns[1:, 0].astype(jnp.int32) + off).reshape(-1)
    n_dst = (ns[1:, 1].astype(jnp.int32) + off).reshape(-1)
    src_all = jnp.concatenate([p_src, n_src])
    dst_all = jnp.concatenate([p_dst, n_dst])
    table = zs[:-1].reshape(s * n, d)
    out = _make_sc_decode(2 * s * e, s * n, d)(src_all, dst_all, table)
    return out[: s * e], out[s * e:]


# speedup vs baseline: 4.4378x; 3.8634x over previous
import functools

import jax
import jax.numpy as jnp
from jax import lax
from jax.experimental import pallas as pl
from jax.experimental.pallas import tpu as pltpu
from jax.experimental.pallas import tpu_sc as plsc

_NC, _NS, _L = 2, 16, 16
_NW = _NC * _NS
_B = 400


def _make_sc_decode(n_edges, n_rows, d):
    n_per_w = n_edges // _NW
    n_chunks = n_per_w // _B
    assert n_per_w % _B == 0 and _B % _L == 0
    n_groups = _B // _L
    mesh = plsc.VectorSubcoreMesh(core_axis_name="c", subcore_axis_name="s")

    @functools.partial(
        pl.kernel,
        out_type=jax.ShapeDtypeStruct((n_edges,), jnp.float32),
        mesh=mesh,
        scratch_types=[
            pltpu.VMEM((_B,), jnp.int32),
            pltpu.VMEM((_B,), jnp.int32),
            pltpu.VMEM((_B, d), jnp.float32),
            pltpu.VMEM((_B, d), jnp.float32),
            pltpu.VMEM((_B,), jnp.float32),
            pltpu.VMEM((_L * _L,), jnp.float32),
            pltpu.SemaphoreType.DMA,
            pltpu.SemaphoreType.DMA,
        ],
        compiler_params=pltpu.CompilerParams(needs_layout_passes=False),
    )
    def decode(src_hbm, dst_hbm, table_hbm, out_hbm,
               idx_s, idx_d, rows_s, rows_d, out_v, stage, sem_s, sem_d):
        wid = lax.axis_index("s") * _NC + lax.axis_index("c")
        wbase = wid * n_per_w
        lanes16 = lax.iota(jnp.int32, _L) * _L

        def chunk_body(c, carry):
            base = wbase + c * _B
            pltpu.sync_copy(src_hbm.at[pl.ds(base, _B)], idx_s)
            pltpu.sync_copy(dst_hbm.at[pl.ds(base, _B)], idx_d)
            cp_s = pltpu.async_copy(table_hbm.at[idx_s], rows_s, sem_s)
            cp_d = pltpu.async_copy(table_hbm.at[idx_d], rows_d, sem_d)
            cp_s.wait()
            cp_d.wait()

            def group_body(j, carry2):
                e0 = j * _L
                for e in range(_L):
                    eid = e0 + e
                    parts = [
                        rows_s[eid, pl.ds(k * _L, _L)]
                        * rows_d[eid, pl.ds(k * _L, _L)]
                        for k in range(d // _L)
                    ]
                    while len(parts) > 1:
                        parts = [
                            parts[i] + parts[i + 1]
                            for i in range(0, len(parts) - 1, 2)
                        ] + ([parts[-1]] if len(parts) % 2 else [])
                    plsc.store_scatter(stage, [lanes16 + e], parts[0])
                dots = stage[pl.ds(0, _L)]
                for l in range(1, _L):
                    dots = dots + stage[pl.ds(l * _L, _L)]
                out_v[pl.ds(e0, _L)] = 1.0 / (1.0 + jnp.exp(-dots))
                return carry2

            lax.fori_loop(0, n_groups, group_body, 0)
            pltpu.sync_copy(out_v, out_hbm.at[pl.ds(base, _B)])
            return carry

        lax.fori_loop(0, n_chunks, chunk_body, 0)

    return decode


def kernel(ps, ns, zs):
    t, n, d = zs.shape
    e = ps.shape[2]
    s = t - 1
    off = (jnp.arange(s, dtype=jnp.int32) * n)[:, None]
    p_src = (ps[1:, 0].astype(jnp.int32) + off).reshape(-1)
    p_dst = (ps[1:, 1].astype(jnp.int32) + off).reshape(-1)
    n_src = (ns[1:, 0].astype(jnp.int32) + off).reshape(-1)
    n_dst = (ns[1:, 1].astype(jnp.int32) + off).reshape(-1)
    src_all = jnp.concatenate([p_src, n_src])
    dst_all = jnp.concatenate([p_dst, n_dst])
    table = zs[:-1].reshape(s * n, d)
    out = _make_sc_decode(2 * s * e, s * n, d)(src_all, dst_all, table)
    return out[: s * e], out[s * e:]
